# baseline (speedup 1.0000x reference)
import functools

import jax
import jax.numpy as jnp
from jax import lax
from jax.experimental import pallas as pl
from jax.experimental.pallas import tpu as pltpu
from jax.experimental.pallas import tpu_sc as plsc

_NC = 2
_NS = 16
_L = 16
_NW = _NC * _NS


def _tc_logits_body(x_ref, wt_ref, b_ref, out_ref):
    p = jnp.dot(x_ref[...], wt_ref[...], preferred_element_type=jnp.float32)
    out_ref[...] = p.T + b_ref[...]


def _tc_fused_body(x_ref, wt_ref, b_ref, lt_ref, score_ref, idx_ref):
    p = jnp.dot(x_ref[...], wt_ref[...], preferred_element_type=jnp.float32)
    p = p + b_ref[...]
    lt_ref[...] = p.T
    e = p.shape[1]
    m = jnp.max(p, axis=1, keepdims=True)
    ii = lax.broadcasted_iota(jnp.int32, p.shape, 1)
    idx_ref[...] = jnp.min(jnp.where(p == m, ii, e), axis=1, keepdims=True)
    score_ref[...] = 1.0 / jnp.sum(jnp.exp(p - m), axis=1, keepdims=True)


def _sc_top1_body(lt_hbm, score_hbm, idx_hbm, lbuf, sbuf, ibuf):
    E = lbuf.shape[0]
    tpw = lbuf.shape[1]
    n_sc = score_hbm.shape[0]
    sc0 = lt_hbm.shape[1] - n_sc
    wid = lax.axis_index("s") * _NC + lax.axis_index("c")
    pltpu.sync_copy(lt_hbm.at[:, pl.ds(sc0 + wid * tpw, tpw)], lbuf)

    def step(j, _):
        off = j * _L
        ls = [lbuf[e, pl.ds(off, _L)] for e in range(E)]
        m = ls[0]
        idx = jnp.zeros((_L,), jnp.int32)
        for e in range(1, E):
            g = ls[e] > m
            m = jnp.where(g, ls[e], m)
            idx = jnp.where(g, jnp.full((_L,), e, jnp.int32), idx)
        s = jnp.exp(ls[0] - m)
        for e in range(1, E):
            s = s + jnp.exp(ls[e] - m)
        sbuf[pl.ds(off, _L)] = 1.0 / s
        ibuf[pl.ds(off, _L)] = idx
        return 0

    lax.fori_loop(0, tpw // _L, step, 0)
    base = wid * tpw
    pltpu.sync_copy(sbuf, score_hbm.at[pl.ds(base, tpw)])
    pltpu.sync_copy(ibuf, idx_hbm.at[pl.ds(base, tpw)])


def kernel(x, W, b):
    d_model = x.shape[-1]
    n_experts = W.shape[0]
    x_flat = x.reshape(-1, d_model)
    n_tok = x_flat.shape[0]
    n_sc = 8192
    n_tc = n_tok - n_sc
    tpw = n_sc // _NW
    bm = 4096

    logits_t, s_tc, i_tc = pl.pallas_call(
        _tc_fused_body,
        grid=(n_tok // bm,),
        in_specs=[
            pl.BlockSpec((bm, d_model), lambda i: (i, 0)),
            pl.BlockSpec((d_model, n_experts), lambda i: (0, 0)),
            pl.BlockSpec((1, n_experts), lambda i: (0, 0)),
        ],
        out_specs=[
            pl.BlockSpec((n_experts, bm), lambda i: (0, i)),
            pl.BlockSpec((bm, 1), lambda i: (i, 0)),
            pl.BlockSpec((bm, 1), lambda i: (i, 0)),
        ],
        out_shape=[
            jax.ShapeDtypeStruct((n_experts, n_tok), jnp.float32),
            jax.ShapeDtypeStruct((n_tok, 1), jnp.float32),
            jax.ShapeDtypeStruct((n_tok, 1), jnp.int32),
        ],
    )(x_flat, W.T, b.reshape(1, n_experts))

    mesh = plsc.VectorSubcoreMesh(core_axis_name="c", subcore_axis_name="s")
    s_sc, i_sc = pl.kernel(
        _sc_top1_body,
        out_type=(
            jax.ShapeDtypeStruct((n_sc,), jnp.float32),
            jax.ShapeDtypeStruct((n_sc,), jnp.int32),
        ),
        mesh=mesh,
        scratch_types=[
            pltpu.VMEM((n_experts, tpw), jnp.float32),
            pltpu.VMEM((tpw,), jnp.float32),
            pltpu.VMEM((tpw,), jnp.int32),
        ],
    )(logits_t)

    scores = jnp.concatenate([s_tc[:n_tc], s_sc.reshape(n_sc, 1)], axis=0)
    idx = jnp.concatenate([i_tc[:n_tc], i_sc.reshape(n_sc, 1)], axis=0)
    return scores, idx


# speedup vs baseline: 1.8099x; 1.8099x over previous
import functools

import jax
import jax.numpy as jnp
from jax import lax
from jax.experimental import pallas as pl
from jax.experimental.pallas import tpu as pltpu
from jax.experimental.pallas import tpu_sc as plsc

_NC = 2
_NS = 16
_L = 16
_NW = _NC * _NS


def _tc_logits_body(x_ref, wt_ref, b_ref, out_ref):
    p = jnp.dot(x_ref[...], wt_ref[...], preferred_element_type=jnp.float32)
    out_ref[...] = p.T + b_ref[...]


def _sc_top1_body(lt_hbm, score_hbm, idx_hbm, lbuf, sbuf, ibuf):
    E = lbuf.shape[0]
    tpw = lbuf.shape[1]
    wid = lax.axis_index("s") * _NC + lax.axis_index("c")
    pltpu.sync_copy(lt_hbm.at[:, pl.ds(wid * tpw, tpw)], lbuf)

    def step(j, _):
        off = j * _L
        ls = [lbuf[e, pl.ds(off, _L)] for e in range(E)]
        m = ls[0]
        idx = jnp.zeros((_L,), jnp.int32)
        for e in range(1, E):
            g = ls[e] > m
            m = jnp.where(g, ls[e], m)
            idx = jnp.where(g, jnp.full((_L,), e, jnp.int32), idx)
        s = jnp.exp(ls[0] - m)
        for e in range(1, E):
            s = s + jnp.exp(ls[e] - m)
        sbuf[pl.ds(off, _L)] = 1.0 / s
        ibuf[pl.ds(off, _L)] = idx
        return 0

    lax.fori_loop(0, tpw // _L, step, 0)
    base = wid * tpw
    pltpu.sync_copy(sbuf, score_hbm.at[pl.ds(base, tpw)])
    pltpu.sync_copy(ibuf, idx_hbm.at[pl.ds(base, tpw)])


def kernel(x, W, b):
    d_model = x.shape[-1]
    n_experts = W.shape[0]
    x_flat = x.reshape(-1, d_model)
    n_tok = x_flat.shape[0]
    tpw = n_tok // _NW
    bm = 4096

    logits_t = pl.pallas_call(
        _tc_logits_body,
        grid=(n_tok // bm,),
        in_specs=[
            pl.BlockSpec((bm, d_model), lambda i: (i, 0)),
            pl.BlockSpec((d_model, n_experts), lambda i: (0, 0)),
            pl.BlockSpec((n_experts, 1), lambda i: (0, 0)),
        ],
        out_specs=pl.BlockSpec((n_experts, bm), lambda i: (0, i)),
        out_shape=jax.ShapeDtypeStruct((n_experts, n_tok), jnp.float32),
    )(x_flat, W.T, b.reshape(n_experts, 1))

    mesh = plsc.VectorSubcoreMesh(core_axis_name="c", subcore_axis_name="s")
    scores, idx = pl.kernel(
        _sc_top1_body,
        out_type=(
            jax.ShapeDtypeStruct((n_tok,), jnp.float32),
            jax.ShapeDtypeStruct((n_tok,), jnp.int32),
        ),
        mesh=mesh,
        scratch_types=[
            pltpu.VMEM((n_experts, tpw), jnp.float32),
            pltpu.VMEM((tpw,), jnp.float32),
            pltpu.VMEM((tpw,), jnp.int32),
        ],
    )(logits_t)

    return scores.reshape(n_tok, 1), idx.reshape(n_tok, 1)
